# baseline (speedup 1.0000x reference)
import functools

import jax
import jax.numpy as jnp
from jax import lax
from jax.experimental import pallas as pl
from jax.experimental.pallas import tpu as pltpu
from jax.experimental.pallas import tpu_sc as plsc

_N = 10000
_E = 320000
_ND = 128
_HD = 256
_ED = 16
_NRBF = 16
_CUTOFF = 10.0
_XP = 16
_HX = _ND + _XP

_CH = 128
_NCHUNK = _E // _CH
_NC = 2
_NS = 16
_NW = _NC * _NS
_NPS = _N // _NS

_BE = 1280
_BN = 1000

@functools.lru_cache(maxsize=None)
def _sc_mesh():
    return plsc.VectorSubcoreMesh(
        core_axis_name="c", subcore_axis_name="s", num_cores=_NC, num_subcores=_NS
    )


def _silu(v):
    return v * jax.nn.sigmoid(v)


def _sc_gather_body(hx_hbm, ii_hbm, jj_hbm,
                    hxi_hbm, hxj_hbm,
                    idx_i, idx_j, bi, bj, sem):
    c = lax.axis_index("c")
    s = lax.axis_index("s")
    wid = s * _NC + c
    trips = (_NCHUNK + _NW - 1) // _NW

    def body(t, carry):
        chunk = wid + t * _NW

        @pl.when(chunk < _NCHUNK)
        def _():
            base = chunk * _CH
            pltpu.sync_copy(ii_hbm.at[pl.ds(base, _CH)], idx_i)
            pltpu.sync_copy(jj_hbm.at[pl.ds(base, _CH)], idx_j)
            c1 = pltpu.async_copy(hx_hbm.at[idx_i], bi, sem)
            c2 = pltpu.async_copy(hx_hbm.at[idx_j], bj, sem)
            c1.wait()
            c2.wait()
            pltpu.sync_copy(bi, hxi_hbm.at[pl.ds(base, _CH)])
            pltpu.sync_copy(bj, hxj_hbm.at[pl.ds(base, _CH)])

        return carry

    lax.fori_loop(0, trips, body, 0)


@functools.lru_cache(maxsize=None)
def _gather_kernel():
    return pl.kernel(
        _sc_gather_body,
        out_type=(
            jax.ShapeDtypeStruct((_E, _HX), jnp.float32),
            jax.ShapeDtypeStruct((_E, _HX), jnp.float32),
        ),
        mesh=_sc_mesh(),
        compiler_params=pltpu.CompilerParams(use_tc_tiling_on_sc=False),
        scratch_types=[
            pltpu.VMEM((_CH,), jnp.int32),
            pltpu.VMEM((_CH,), jnp.int32),
            pltpu.VMEM((_CH, _HX), jnp.float32),
            pltpu.VMEM((_CH, _HX), jnp.float32),
            pltpu.SemaphoreType.DMA,
        ],
    )


def _gather_call(hx, ii, jj):
    return _gather_kernel()(hx, ii, jj)


def _sc_scatter_body(m0_hbm, m1_hbm, ii_hbm, z_hbm,
                     s0_hbm, s1_hbm,
                     idx_v, mbuf, shm, sem):
    c = lax.axis_index("c")
    s = lax.axis_index("s")
    rows = pl.ds(s * _NPS, _NPS)
    pltpu.sync_copy(z_hbm, shm.at[rows])
    plsc.subcore_barrier()

    trips = (_NCHUNK + _NS - 1) // _NS

    def body(t, carry):
        chunk = s + t * _NS

        @pl.when(chunk < _NCHUNK)
        def _():
            base = chunk * _CH
            pltpu.sync_copy(ii_hbm.at[pl.ds(base, _CH)], idx_v)

            @pl.when(c == 0)
            def _():
                pltpu.sync_copy(m0_hbm.at[pl.ds(base, _CH)], mbuf)

            @pl.when(c == 1)
            def _():
                pltpu.sync_copy(m1_hbm.at[pl.ds(base, _CH)], mbuf)

            pltpu.sync_copy(mbuf, shm.at[idx_v], add=True)

        return carry

    lax.fori_loop(0, trips, body, 0)
    plsc.subcore_barrier()

    @pl.when(c == 0)
    def _():
        pltpu.sync_copy(shm.at[rows], s0_hbm.at[rows])

    @pl.when(c == 1)
    def _():
        pltpu.sync_copy(shm.at[rows], s1_hbm.at[rows])


@functools.lru_cache(maxsize=None)
def _scatter_kernel():
    return pl.kernel(
        _sc_scatter_body,
        out_type=(
            jax.ShapeDtypeStruct((_N, _HX), jnp.float32),
            jax.ShapeDtypeStruct((_N, _HX), jnp.float32),
        ),
        mesh=_sc_mesh(),
        compiler_params=pltpu.CompilerParams(use_tc_tiling_on_sc=False),
        scratch_types=[
            pltpu.VMEM((_CH,), jnp.int32),
            pltpu.VMEM((_CH, _HX), jnp.float32),
            pltpu.VMEM_SHARED((_N, _HX), jnp.float32),
            pltpu.SemaphoreType.DMA,
        ],
    )


def _scatter_call(m0, m1, ii, z):
    return _scatter_kernel()(m0, m1, ii, z)


def _tc_edge_body(hxi, hxj, ea,
                  w1, b1, w2, b2, wx1, bx1, wx2,
                  m0_o, m1_o):
    f32 = jnp.float32
    di = hxi[:, _ND:] - hxj[:, _ND:]
    d2 = jnp.sum(di * di, axis=1, keepdims=True) + 1e-8
    dist = jnp.sqrt(d2)
    centers = lax.broadcasted_iota(jnp.int32, (1, _NRBF), 1).astype(f32) * (
        _CUTOFF / (_NRBF - 1))
    zz = (dist - centers) * (_NRBF / _CUTOFF)
    rbf = jnp.exp(-0.5 * zz * zz)
    msg = jnp.concatenate([hxi[:, :_ND], hxj[:, :_ND], rbf, ea[...]], axis=1)
    pre = jnp.dot(msg, w1[...], preferred_element_type=f32) + b1[...]
    m = _silu(pre)
    m = _silu(jnp.dot(m, w2[...], preferred_element_type=f32) + b2[...])
    t = _silu(jnp.dot(m, wx1[...], preferred_element_type=f32) + bx1[...])
    cw = jnp.dot(t, wx2[...], preferred_element_type=f32)
    m0_o[...] = jnp.concatenate([m[:, :_ND], di * cw], axis=1)
    m1_o[...] = jnp.concatenate(
        [m[:, _ND:], jnp.zeros((m.shape[0], _XP), f32)], axis=1)


def _edge_call(hxi, hxj, ea, w1, b1, w2, b2, wx1, bx1, wx2):
    grid = (_E // _BE,)
    row = lambda i: (i, 0)
    full = lambda i: (0, 0)
    return pl.pallas_call(
        _tc_edge_body,
        grid=grid,
        in_specs=[
            pl.BlockSpec((_BE, _HX), row),
            pl.BlockSpec((_BE, _HX), row),
            pl.BlockSpec((_BE, _ED), row),
            pl.BlockSpec((2 * _ND + _NRBF + _ED, _HD), full),
            pl.BlockSpec((1, _HD), full),
            pl.BlockSpec((_HD, _HD), full),
            pl.BlockSpec((1, _HD), full),
            pl.BlockSpec((_HD, _HD), full),
            pl.BlockSpec((1, _HD), full),
            pl.BlockSpec((_HD, 1), full),
        ],
        out_specs=[
            pl.BlockSpec((_BE, _HX), row),
            pl.BlockSpec((_BE, _HX), row),
        ],
        out_shape=[
            jax.ShapeDtypeStruct((_E, _HX), jnp.float32),
            jax.ShapeDtypeStruct((_E, _HX), jnp.float32),
        ],
        compiler_params=pltpu.CompilerParams(
            dimension_semantics=("arbitrary",),
        ),
    )(hxi, hxj, ea, w1, b1, w2, b2, wx1, bx1, wx2)


def _tc_node_body(hx, s0, s1,
                  wh1, bh1, wh2, bh2, g, b,
                  hx_o):
    f32 = jnp.float32
    hv = hx[:, :_ND]
    cat = jnp.concatenate([hv, s0[:, :_ND], s1[:, :_ND]], axis=1)
    pre = jnp.dot(cat, wh1[...], preferred_element_type=f32) + bh1[...]
    u = jnp.dot(_silu(pre), wh2[...], preferred_element_type=f32) + bh2[...]
    hn = hv + u
    mu = jnp.mean(hn, axis=1, keepdims=True)
    var = jnp.mean((hn - mu) * (hn - mu), axis=1, keepdims=True)
    hnorm = (hn - mu) * lax.rsqrt(var + 1e-5) * g[...] + b[...]
    xnew = hx[:, _ND:] + s0[:, _ND:]
    hx_o[...] = jnp.concatenate([hnorm, xnew], axis=1)


def _node_call(hx, s0, s1, wh1, bh1, wh2, bh2, g, b):
    grid = (_N // _BN,)
    row = lambda i: (i, 0)
    full = lambda i: (0, 0)
    return pl.pallas_call(
        _tc_node_body,
        grid=grid,
        in_specs=[
            pl.BlockSpec((_BN, _HX), row),
            pl.BlockSpec((_BN, _HX), row),
            pl.BlockSpec((_BN, _HX), row),
            pl.BlockSpec((_ND + _HD, _HD), full),
            pl.BlockSpec((1, _HD), full),
            pl.BlockSpec((_HD, _ND), full),
            pl.BlockSpec((1, _ND), full),
            pl.BlockSpec((1, _ND), full),
            pl.BlockSpec((1, _ND), full),
        ],
        out_specs=[
            pl.BlockSpec((_BN, _HX), row),
        ],
        out_shape=[
            jax.ShapeDtypeStruct((_N, _HX), jnp.float32),
        ],
        compiler_params=pltpu.CompilerParams(
            dimension_semantics=("arbitrary",),
        ),
    )(hx, s0, s1, wh1, bh1, wh2, bh2, g, b)


def kernel(h, x, edge_index, edge_attr, params):
    ei = edge_index.astype(jnp.int32)
    ii = ei[1]
    jj = ei[0]
    xp = jnp.pad(x.astype(jnp.float32), ((0, 0), (0, _XP - 3)))
    hx = jnp.concatenate([h, xp], axis=1)
    z = jnp.zeros((_NPS, _HX), jnp.float32)
    for p in params:
        hxi, hxj = _gather_call(hx, ii, jj)
        m0, m1 = _edge_call(
            hxi, hxj, edge_attr,
            p["e1"]["w"], p["e1"]["b"][None],
            p["e2"]["w"], p["e2"]["b"][None],
            p["x1"]["w"], p["x1"]["b"][None], p["x2"]["w"],
        )
        s0, s1 = _scatter_call(m0, m1, ii, z)
        (hx,) = _node_call(
            hx, s0, s1,
            p["h1"]["w"], p["h1"]["b"][None],
            p["h2"]["w"], p["h2"]["b"][None], p["ln_g"][None], p["ln_b"][None],
        )
    return (hx[:, :_ND], hx[:, _ND:_ND + 3])


# speedup vs baseline: 1.7482x; 1.7482x over previous
import functools

import jax
import jax.numpy as jnp
from jax import lax
from jax.experimental import pallas as pl
from jax.experimental.pallas import tpu as pltpu
from jax.experimental.pallas import tpu_sc as plsc

_N = 10000
_E = 320000
_ND = 128
_HD = 256
_ED = 16
_NRBF = 16
_CUTOFF = 10.0
_XP = 16

_S = 2
_ES = _E // _S
_CH = 128
_NCHS = _ES // _CH
_NC = 2
_NS = 16
_NW = _NC * _NS
_NPS = _N // _NS

_BE = 1280
_BN = 1000

@functools.lru_cache(maxsize=None)
def _sc_mesh():
    return plsc.VectorSubcoreMesh(
        core_axis_name="c", subcore_axis_name="s", num_cores=_NC, num_subcores=_NS
    )


def _silu(v):
    return v * jax.nn.sigmoid(v)


def _sc_gather_body(h_hbm, xp_hbm, ii_hbm, jj_hbm,
                    hi_hbm, hj_hbm, xi_hbm, xj_hbm,
                    idx_i, idx_j, bhi, bhj, bxi, bxj, sem):
    c = lax.axis_index("c")
    s = lax.axis_index("s")
    wid = s * _NC + c
    trips = (_NCHS + _NW - 1) // _NW

    def body(t, carry):
        chunk = wid + t * _NW

        @pl.when(chunk < _NCHS)
        def _():
            base = chunk * _CH
            pltpu.sync_copy(ii_hbm.at[pl.ds(base, _CH)], idx_i)
            pltpu.sync_copy(jj_hbm.at[pl.ds(base, _CH)], idx_j)
            c1 = pltpu.async_copy(h_hbm.at[idx_i], bhi, sem)
            c2 = pltpu.async_copy(h_hbm.at[idx_j], bhj, sem)
            c3 = pltpu.async_copy(xp_hbm.at[idx_i], bxi, sem)
            c4 = pltpu.async_copy(xp_hbm.at[idx_j], bxj, sem)
            c1.wait()
            c2.wait()
            c3.wait()
            c4.wait()
            pltpu.sync_copy(bhi, hi_hbm.at[pl.ds(base, _CH)])
            pltpu.sync_copy(bhj, hj_hbm.at[pl.ds(base, _CH)])
            pltpu.sync_copy(bxi, xi_hbm.at[pl.ds(base, _CH)])
            pltpu.sync_copy(bxj, xj_hbm.at[pl.ds(base, _CH)])

        return carry

    lax.fori_loop(0, trips, body, 0)


@functools.lru_cache(maxsize=None)
def _gather_kernel():
    return pl.kernel(
        _sc_gather_body,
        out_type=(
            jax.ShapeDtypeStruct((_ES, _ND), jnp.float32),
            jax.ShapeDtypeStruct((_ES, _ND), jnp.float32),
            jax.ShapeDtypeStruct((_ES, _XP), jnp.float32),
            jax.ShapeDtypeStruct((_ES, _XP), jnp.float32),
        ),
        mesh=_sc_mesh(),
        compiler_params=pltpu.CompilerParams(use_tc_tiling_on_sc=False),
        scratch_types=[
            pltpu.VMEM((_CH,), jnp.int32),
            pltpu.VMEM((_CH,), jnp.int32),
            pltpu.VMEM((_CH, _ND), jnp.float32),
            pltpu.VMEM((_CH, _ND), jnp.float32),
            pltpu.VMEM((_CH, _XP), jnp.float32),
            pltpu.VMEM((_CH, _XP), jnp.float32),
            pltpu.SemaphoreType.DMA,
        ],
    )


def _gather_call(h, xp, ii, jj):
    return _gather_kernel()(h, xp, ii, jj)


def _sc_scatter_body(mlo_hbm, mhi_hbm, wd_hbm, ii_hbm, ilo_hbm, ihi_hbm,
                     ix_hbm,
                     silo_hbm, sihi_hbm, xacc_hbm,
                     idx_v, mbuf, wbuf, shm, shx, sem):
    c = lax.axis_index("c")
    s = lax.axis_index("s")
    rows = pl.ds(s * _NPS, _NPS)

    @pl.when(c == 0)
    def _():
        pltpu.sync_copy(ilo_hbm.at[rows], shm.at[rows])
        pltpu.sync_copy(ix_hbm.at[rows], shx.at[rows])

    @pl.when(c == 1)
    def _():
        pltpu.sync_copy(ihi_hbm.at[rows], shm.at[rows])

    plsc.subcore_barrier()

    trips = (_NCHS + _NS - 1) // _NS

    def body(t, carry):
        chunk = s + t * _NS

        @pl.when(chunk < _NCHS)
        def _():
            base = chunk * _CH
            pltpu.sync_copy(ii_hbm.at[pl.ds(base, _CH)], idx_v)

            @pl.when(c == 0)
            def _():
                pltpu.sync_copy(mlo_hbm.at[pl.ds(base, _CH)], mbuf)
                pltpu.sync_copy(wd_hbm.at[pl.ds(base, _CH)], wbuf)
                pltpu.sync_copy(mbuf, shm.at[idx_v], add=True)
                pltpu.sync_copy(wbuf, shx.at[idx_v], add=True)

            @pl.when(c == 1)
            def _():
                pltpu.sync_copy(mhi_hbm.at[pl.ds(base, _CH)], mbuf)
                pltpu.sync_copy(mbuf, shm.at[idx_v], add=True)

        return carry

    lax.fori_loop(0, trips, body, 0)
    plsc.subcore_barrier()

    @pl.when(c == 0)
    def _():
        pltpu.sync_copy(shm.at[rows], silo_hbm.at[rows])
        pltpu.sync_copy(shx.at[rows], xacc_hbm.at[rows])

    @pl.when(c == 1)
    def _():
        pltpu.sync_copy(shm.at[rows], sihi_hbm.at[rows])


@functools.lru_cache(maxsize=None)
def _scatter_kernel():
    return pl.kernel(
        _sc_scatter_body,
        out_type=(
            jax.ShapeDtypeStruct((_N, _ND), jnp.float32),
            jax.ShapeDtypeStruct((_N, _ND), jnp.float32),
            jax.ShapeDtypeStruct((_N, _XP), jnp.float32),
        ),
        mesh=_sc_mesh(),
        compiler_params=pltpu.CompilerParams(use_tc_tiling_on_sc=False),
        scratch_types=[
            pltpu.VMEM((_CH,), jnp.int32),
            pltpu.VMEM((_CH, _ND), jnp.float32),
            pltpu.VMEM((_CH, _XP), jnp.float32),
            pltpu.VMEM_SHARED((_N, _ND), jnp.float32),
            pltpu.VMEM_SHARED((_N, _XP), jnp.float32),
            pltpu.SemaphoreType.DMA,
        ],
    )


def _scatter_call(mlo, mhi, wd, ii, ilo, ihi, ix):
    return _scatter_kernel()(mlo, mhi, wd, ii, ilo, ihi, ix)


def _tc_edge_body(hi, hj, xi, xj, ea,
                  w1, b1, w2, b2, wx1, bx1, wx2,
                  mlo_o, mhi_o, wd_o):
    f32 = jnp.float32
    di = xi[...] - xj[...]
    d2 = jnp.sum(di * di, axis=1, keepdims=True) + 1e-8
    dist = jnp.sqrt(d2)
    centers = lax.broadcasted_iota(jnp.int32, (1, _NRBF), 1).astype(f32) * (
        _CUTOFF / (_NRBF - 1))
    zz = (dist - centers) * (_NRBF / _CUTOFF)
    rbf = jnp.exp(-0.5 * zz * zz)
    msg = jnp.concatenate([hi[...], hj[...], rbf, ea[...]], axis=1)
    pre = jnp.dot(msg, w1[...], preferred_element_type=f32) + b1[...]
    m = _silu(pre)
    m = _silu(jnp.dot(m, w2[...], preferred_element_type=f32) + b2[...])
    t = _silu(jnp.dot(m, wx1[...], preferred_element_type=f32) + bx1[...])
    cw = jnp.dot(t, wx2[...], preferred_element_type=f32)
    mlo_o[...] = m[:, :_ND]
    mhi_o[...] = m[:, _ND:]
    wd_o[...] = di * cw


def _edge_call(hi, hj, xi, xj, ea, w1, b1, w2, b2, wx1, bx1, wx2):
    grid = (_ES // _BE,)
    row = lambda i: (i, 0)
    full = lambda i: (0, 0)
    return pl.pallas_call(
        _tc_edge_body,
        grid=grid,
        in_specs=[
            pl.BlockSpec((_BE, _ND), row),
            pl.BlockSpec((_BE, _ND), row),
            pl.BlockSpec((_BE, _XP), row),
            pl.BlockSpec((_BE, _XP), row),
            pl.BlockSpec((_BE, _ED), row),
            pl.BlockSpec((2 * _ND + _NRBF + _ED, _HD), full),
            pl.BlockSpec((1, _HD), full),
            pl.BlockSpec((_HD, _HD), full),
            pl.BlockSpec((1, _HD), full),
            pl.BlockSpec((_HD, _HD), full),
            pl.BlockSpec((1, _HD), full),
            pl.BlockSpec((_HD, 1), full),
        ],
        out_specs=[
            pl.BlockSpec((_BE, _ND), row),
            pl.BlockSpec((_BE, _ND), row),
            pl.BlockSpec((_BE, _XP), row),
        ],
        out_shape=[
            jax.ShapeDtypeStruct((_ES, _ND), jnp.float32),
            jax.ShapeDtypeStruct((_ES, _ND), jnp.float32),
            jax.ShapeDtypeStruct((_ES, _XP), jnp.float32),
        ],
        compiler_params=pltpu.CompilerParams(
            dimension_semantics=("arbitrary",),
        ),
    )(hi, hj, xi, xj, ea, w1, b1, w2, b2, wx1, bx1, wx2)


def _tc_node_body(h, mlo, mhi, xp, xacc,
                  wh1, bh1, wh2, bh2, g, b,
                  hn_o, xp_o):
    f32 = jnp.float32
    hv = h[...]
    cat = jnp.concatenate([hv, mlo[...], mhi[...]], axis=1)
    pre = jnp.dot(cat, wh1[...], preferred_element_type=f32) + bh1[...]
    u = jnp.dot(_silu(pre), wh2[...], preferred_element_type=f32) + bh2[...]
    hn = hv + u
    mu = jnp.mean(hn, axis=1, keepdims=True)
    var = jnp.mean((hn - mu) * (hn - mu), axis=1, keepdims=True)
    hn_o[...] = (hn - mu) * lax.rsqrt(var + 1e-5) * g[...] + b[...]
    xp_o[...] = xp[...] + xacc[...]


def _node_call(h, mlo, mhi, xp, xacc, wh1, bh1, wh2, bh2, g, b):
    grid = (_N // _BN,)
    row = lambda i: (i, 0)
    full = lambda i: (0, 0)
    return pl.pallas_call(
        _tc_node_body,
        grid=grid,
        in_specs=[
            pl.BlockSpec((_BN, _ND), row),
            pl.BlockSpec((_BN, _ND), row),
            pl.BlockSpec((_BN, _ND), row),
            pl.BlockSpec((_BN, _XP), row),
            pl.BlockSpec((_BN, _XP), row),
            pl.BlockSpec((_ND + _HD, _HD), full),
            pl.BlockSpec((1, _HD), full),
            pl.BlockSpec((_HD, _ND), full),
            pl.BlockSpec((1, _ND), full),
            pl.BlockSpec((1, _ND), full),
            pl.BlockSpec((1, _ND), full),
        ],
        out_specs=[
            pl.BlockSpec((_BN, _ND), row),
            pl.BlockSpec((_BN, _XP), row),
        ],
        out_shape=[
            jax.ShapeDtypeStruct((_N, _ND), jnp.float32),
            jax.ShapeDtypeStruct((_N, _XP), jnp.float32),
        ],
        compiler_params=pltpu.CompilerParams(
            dimension_semantics=("arbitrary",),
        ),
    )(h, mlo, mhi, xp, xacc, wh1, bh1, wh2, bh2, g, b)


def kernel(h, x, edge_index, edge_attr, params):
    ei = edge_index.astype(jnp.int32)
    iis = [lax.slice(ei[1], (k * _ES,), ((k + 1) * _ES,)) for k in range(_S)]
    jjs = [lax.slice(ei[0], (k * _ES,), ((k + 1) * _ES,)) for k in range(_S)]
    eas = [lax.slice(edge_attr, (k * _ES, 0), ((k + 1) * _ES, _ED))
           for k in range(_S)]
    xp = jnp.pad(x.astype(jnp.float32), ((0, 0), (0, _XP - 3)))
    z = jnp.zeros((_N, _ND), jnp.float32)
    zx = jnp.zeros((_N, _XP), jnp.float32)
    for p in params:
        silo, sihi, xacc = z, z, zx
        for k in range(_S):
            hi, hj, xi, xj = _gather_call(h, xp, iis[k], jjs[k])
            mlo, mhi, wd = _edge_call(
                hi, hj, xi, xj, eas[k],
                p["e1"]["w"], p["e1"]["b"][None],
                p["e2"]["w"], p["e2"]["b"][None],
                p["x1"]["w"], p["x1"]["b"][None], p["x2"]["w"],
            )
            silo, sihi, xacc = _scatter_call(
                mlo, mhi, wd, iis[k], silo, sihi, xacc)
        h, xp = _node_call(
            h, silo, sihi, xp, xacc,
            p["h1"]["w"], p["h1"]["b"][None],
            p["h2"]["w"], p["h2"]["b"][None], p["ln_g"][None], p["ln_b"][None],
        )
    return (h, xp[:, :3])


# speedup vs baseline: 2.0671x; 1.1824x over previous
import functools

import jax
import jax.numpy as jnp
from jax import lax
from jax.experimental import pallas as pl
from jax.experimental.pallas import tpu as pltpu
from jax.experimental.pallas import tpu_sc as plsc

_N = 10000
_E = 320000
_ND = 128
_HD = 256
_ED = 16
_NRBF = 16
_CUTOFF = 10.0
_XP = 16

_S = 4
_ES = _E // _S
_CH = 128
_NCHS = _ES // _CH
_NC = 2
_NS = 16
_NW = _NC * _NS
_NPS = _N // _NS

_BE = 1600
_BN = 1000

@functools.lru_cache(maxsize=None)
def _sc_mesh():
    return plsc.VectorSubcoreMesh(
        core_axis_name="c", subcore_axis_name="s", num_cores=_NC, num_subcores=_NS
    )


def _silu(v):
    return v * jax.nn.sigmoid(v)


def _sc_gather_body(h_hbm, xp_hbm, ii_hbm, jj_hbm,
                    hi_hbm, hj_hbm, xi_hbm, xj_hbm,
                    idx_i, idx_j, bhi, bhj, bxi, bxj, sem):
    c = lax.axis_index("c")
    s = lax.axis_index("s")
    wid = s * _NC + c
    trips = (_NCHS + _NW - 1) // _NW

    def body(t, carry):
        chunk = wid + t * _NW

        @pl.when(chunk < _NCHS)
        def _():
            base = chunk * _CH
            pltpu.sync_copy(ii_hbm.at[pl.ds(base, _CH)], idx_i)
            pltpu.sync_copy(jj_hbm.at[pl.ds(base, _CH)], idx_j)
            c1 = pltpu.async_copy(h_hbm.at[idx_i], bhi, sem)
            c2 = pltpu.async_copy(h_hbm.at[idx_j], bhj, sem)
            c3 = pltpu.async_copy(xp_hbm.at[idx_i], bxi, sem)
            c4 = pltpu.async_copy(xp_hbm.at[idx_j], bxj, sem)
            c1.wait()
            c2.wait()
            c3.wait()
            c4.wait()
            pltpu.sync_copy(bhi, hi_hbm.at[pl.ds(base, _CH)])
            pltpu.sync_copy(bhj, hj_hbm.at[pl.ds(base, _CH)])
            pltpu.sync_copy(bxi, xi_hbm.at[pl.ds(base, _CH)])
            pltpu.sync_copy(bxj, xj_hbm.at[pl.ds(base, _CH)])

        return carry

    lax.fori_loop(0, trips, body, 0)


@functools.lru_cache(maxsize=None)
def _gather_kernel():
    return pl.kernel(
        _sc_gather_body,
        out_type=(
            jax.ShapeDtypeStruct((_ES, _ND), jnp.float32),
            jax.ShapeDtypeStruct((_ES, _ND), jnp.float32),
            jax.ShapeDtypeStruct((_ES, _XP), jnp.float32),
            jax.ShapeDtypeStruct((_ES, _XP), jnp.float32),
        ),
        mesh=_sc_mesh(),
        compiler_params=pltpu.CompilerParams(use_tc_tiling_on_sc=False),
        scratch_types=[
            pltpu.VMEM((_CH,), jnp.int32),
            pltpu.VMEM((_CH,), jnp.int32),
            pltpu.VMEM((_CH, _ND), jnp.float32),
            pltpu.VMEM((_CH, _ND), jnp.float32),
            pltpu.VMEM((_CH, _XP), jnp.float32),
            pltpu.VMEM((_CH, _XP), jnp.float32),
            pltpu.SemaphoreType.DMA,
        ],
    )


def _gather_call(h, xp, ii, jj):
    return _gather_kernel()(h, xp, ii, jj)


def _sc_scatter_body(mlo_hbm, mhi_hbm, wd_hbm, ii_hbm, ilo_hbm, ihi_hbm,
                     ix_hbm,
                     silo_hbm, sihi_hbm, xacc_hbm,
                     idx_v, mbuf, wbuf, shm, shx, sem):
    c = lax.axis_index("c")
    s = lax.axis_index("s")
    rows = pl.ds(s * _NPS, _NPS)

    @pl.when(c == 0)
    def _():
        pltpu.sync_copy(ilo_hbm.at[rows], shm.at[rows])
        pltpu.sync_copy(ix_hbm.at[rows], shx.at[rows])

    @pl.when(c == 1)
    def _():
        pltpu.sync_copy(ihi_hbm.at[rows], shm.at[rows])

    plsc.subcore_barrier()

    trips = (_NCHS + _NS - 1) // _NS

    def body(t, carry):
        chunk = s + t * _NS

        @pl.when(chunk < _NCHS)
        def _():
            base = chunk * _CH
            pltpu.sync_copy(ii_hbm.at[pl.ds(base, _CH)], idx_v)

            @pl.when(c == 0)
            def _():
                pltpu.sync_copy(mlo_hbm.at[pl.ds(base, _CH)], mbuf)
                pltpu.sync_copy(wd_hbm.at[pl.ds(base, _CH)], wbuf)
                pltpu.sync_copy(mbuf, shm.at[idx_v], add=True)
                pltpu.sync_copy(wbuf, shx.at[idx_v], add=True)

            @pl.when(c == 1)
            def _():
                pltpu.sync_copy(mhi_hbm.at[pl.ds(base, _CH)], mbuf)
                pltpu.sync_copy(mbuf, shm.at[idx_v], add=True)

        return carry

    lax.fori_loop(0, trips, body, 0)
    plsc.subcore_barrier()

    @pl.when(c == 0)
    def _():
        pltpu.sync_copy(shm.at[rows], silo_hbm.at[rows])
        pltpu.sync_copy(shx.at[rows], xacc_hbm.at[rows])

    @pl.when(c == 1)
    def _():
        pltpu.sync_copy(shm.at[rows], sihi_hbm.at[rows])


@functools.lru_cache(maxsize=None)
def _scatter_kernel():
    return pl.kernel(
        _sc_scatter_body,
        out_type=(
            jax.ShapeDtypeStruct((_N, _ND), jnp.float32),
            jax.ShapeDtypeStruct((_N, _ND), jnp.float32),
            jax.ShapeDtypeStruct((_N, _XP), jnp.float32),
        ),
        mesh=_sc_mesh(),
        compiler_params=pltpu.CompilerParams(use_tc_tiling_on_sc=False),
        scratch_types=[
            pltpu.VMEM((_CH,), jnp.int32),
            pltpu.VMEM((_CH, _ND), jnp.float32),
            pltpu.VMEM((_CH, _XP), jnp.float32),
            pltpu.VMEM_SHARED((_N, _ND), jnp.float32),
            pltpu.VMEM_SHARED((_N, _XP), jnp.float32),
            pltpu.SemaphoreType.DMA,
        ],
    )


def _scatter_call(mlo, mhi, wd, ii, ilo, ihi, ix):
    return _scatter_kernel()(mlo, mhi, wd, ii, ilo, ihi, ix)


def _tc_edge_body(hi, hj, xi, xj, ea,
                  w1, b1, w2, b2, wx1, bx1, wx2,
                  mlo_o, mhi_o, wd_o):
    f32 = jnp.float32
    di = xi[...] - xj[...]
    d2 = jnp.sum(di * di, axis=1, keepdims=True) + 1e-8
    dist = jnp.sqrt(d2)
    centers = lax.broadcasted_iota(jnp.int32, (1, _NRBF), 1).astype(f32) * (
        _CUTOFF / (_NRBF - 1))
    zz = (dist - centers) * (_NRBF / _CUTOFF)
    rbf = jnp.exp(-0.5 * zz * zz)
    msg = jnp.concatenate([hi[...], hj[...], rbf, ea[...]], axis=1)
    pre = jnp.dot(msg, w1[...], preferred_element_type=f32) + b1[...]
    m = _silu(pre)
    m = _silu(jnp.dot(m, w2[...], preferred_element_type=f32) + b2[...])
    t = _silu(jnp.dot(m, wx1[...], preferred_element_type=f32) + bx1[...])
    cw = jnp.dot(t, wx2[...], preferred_element_type=f32)
    mlo_o[...] = m[:, :_ND]
    mhi_o[...] = m[:, _ND:]
    wd_o[...] = di * cw


def _edge_call(hi, hj, xi, xj, ea, w1, b1, w2, b2, wx1, bx1, wx2):
    grid = (_ES // _BE,)
    row = lambda i: (i, 0)
    full = lambda i: (0, 0)
    return pl.pallas_call(
        _tc_edge_body,
        grid=grid,
        in_specs=[
            pl.BlockSpec((_BE, _ND), row),
            pl.BlockSpec((_BE, _ND), row),
            pl.BlockSpec((_BE, _XP), row),
            pl.BlockSpec((_BE, _XP), row),
            pl.BlockSpec((_BE, _ED), row),
            pl.BlockSpec((2 * _ND + _NRBF + _ED, _HD), full),
            pl.BlockSpec((1, _HD), full),
            pl.BlockSpec((_HD, _HD), full),
            pl.BlockSpec((1, _HD), full),
            pl.BlockSpec((_HD, _HD), full),
            pl.BlockSpec((1, _HD), full),
            pl.BlockSpec((_HD, 1), full),
        ],
        out_specs=[
            pl.BlockSpec((_BE, _ND), row),
            pl.BlockSpec((_BE, _ND), row),
            pl.BlockSpec((_BE, _XP), row),
        ],
        out_shape=[
            jax.ShapeDtypeStruct((_ES, _ND), jnp.float32),
            jax.ShapeDtypeStruct((_ES, _ND), jnp.float32),
            jax.ShapeDtypeStruct((_ES, _XP), jnp.float32),
        ],
        compiler_params=pltpu.CompilerParams(
            dimension_semantics=("arbitrary",),
        ),
    )(hi, hj, xi, xj, ea, w1, b1, w2, b2, wx1, bx1, wx2)


def _tc_node_body(h, mlo, mhi, xp, xacc,
                  wh1, bh1, wh2, bh2, g, b,
                  hn_o, xp_o):
    f32 = jnp.float32
    hv = h[...]
    cat = jnp.concatenate([hv, mlo[...], mhi[...]], axis=1)
    pre = jnp.dot(cat, wh1[...], preferred_element_type=f32) + bh1[...]
    u = jnp.dot(_silu(pre), wh2[...], preferred_element_type=f32) + bh2[...]
    hn = hv + u
    mu = jnp.mean(hn, axis=1, keepdims=True)
    var = jnp.mean((hn - mu) * (hn - mu), axis=1, keepdims=True)
    hn_o[...] = (hn - mu) * lax.rsqrt(var + 1e-5) * g[...] + b[...]
    xp_o[...] = xp[...] + xacc[...]


def _node_call(h, mlo, mhi, xp, xacc, wh1, bh1, wh2, bh2, g, b):
    grid = (_N // _BN,)
    row = lambda i: (i, 0)
    full = lambda i: (0, 0)
    return pl.pallas_call(
        _tc_node_body,
        grid=grid,
        in_specs=[
            pl.BlockSpec((_BN, _ND), row),
            pl.BlockSpec((_BN, _ND), row),
            pl.BlockSpec((_BN, _ND), row),
            pl.BlockSpec((_BN, _XP), row),
            pl.BlockSpec((_BN, _XP), row),
            pl.BlockSpec((_ND + _HD, _HD), full),
            pl.BlockSpec((1, _HD), full),
            pl.BlockSpec((_HD, _ND), full),
            pl.BlockSpec((1, _ND), full),
            pl.BlockSpec((1, _ND), full),
            pl.BlockSpec((1, _ND), full),
        ],
        out_specs=[
            pl.BlockSpec((_BN, _ND), row),
            pl.BlockSpec((_BN, _XP), row),
        ],
        out_shape=[
            jax.ShapeDtypeStruct((_N, _ND), jnp.float32),
            jax.ShapeDtypeStruct((_N, _XP), jnp.float32),
        ],
        compiler_params=pltpu.CompilerParams(
            dimension_semantics=("arbitrary",),
        ),
    )(h, mlo, mhi, xp, xacc, wh1, bh1, wh2, bh2, g, b)


def kernel(h, x, edge_index, edge_attr, params):
    ei = edge_index.astype(jnp.int32)
    iis = [lax.slice(ei[1], (k * _ES,), ((k + 1) * _ES,)) for k in range(_S)]
    jjs = [lax.slice(ei[0], (k * _ES,), ((k + 1) * _ES,)) for k in range(_S)]
    eas = [lax.slice(edge_attr, (k * _ES, 0), ((k + 1) * _ES, _ED))
           for k in range(_S)]
    xp = jnp.pad(x.astype(jnp.float32), ((0, 0), (0, _XP - 3)))
    z = jnp.zeros((_N, _ND), jnp.float32)
    zx = jnp.zeros((_N, _XP), jnp.float32)
    for p in params:
        silo, sihi, xacc = z, z, zx
        for k in range(_S):
            hi, hj, xi, xj = _gather_call(h, xp, iis[k], jjs[k])
            mlo, mhi, wd = _edge_call(
                hi, hj, xi, xj, eas[k],
                p["e1"]["w"], p["e1"]["b"][None],
                p["e2"]["w"], p["e2"]["b"][None],
                p["x1"]["w"], p["x1"]["b"][None], p["x2"]["w"],
            )
            silo, sihi, xacc = _scatter_call(
                mlo, mhi, wd, iis[k], silo, sihi, xacc)
        h, xp = _node_call(
            h, silo, sihi, xp, xacc,
            p["h1"]["w"], p["h1"]["b"][None],
            p["h2"]["w"], p["h2"]["b"][None], p["ln_g"][None], p["ln_b"][None],
        )
    return (h, xp[:, :3])


# speedup vs baseline: 2.1333x; 1.0320x over previous
import functools

import jax
import jax.numpy as jnp
from jax import lax
from jax.experimental import pallas as pl
from jax.experimental.pallas import tpu as pltpu
from jax.experimental.pallas import tpu_sc as plsc

_N = 10000
_E = 320000
_ND = 128
_HD = 256
_ED = 16
_NRBF = 16
_CUTOFF = 10.0
_XP = 16

_S = 4
_ES = _E // _S
_CH = 128
_NCHS = _ES // _CH
_NC = 2
_NS = 16
_NW = _NC * _NS
_NPS = _N // _NS

_BE = 1600
_BN = 1000

@functools.lru_cache(maxsize=None)
def _sc_mesh():
    return plsc.VectorSubcoreMesh(
        core_axis_name="c", subcore_axis_name="s", num_cores=_NC, num_subcores=_NS
    )


def _silu(v):
    return v * jax.nn.sigmoid(v)


def _sc_gather_body(h_hbm, xp_hbm, ii_hbm, jj_hbm,
                    hi_hbm, hj_hbm, xi_hbm, xj_hbm,
                    ii0, jj0, ii1, jj1,
                    bhi0, bhj0, bxi0, bxj0,
                    bhi1, bhj1, bxi1, bxj1,
                    gsem0, gsem1, wsem0, wsem1):
    c = lax.axis_index("c")
    s = lax.axis_index("s")
    wid = s * _NC + c
    trips = (_NCHS + _NW - 1) // _NW
    pairs = (trips + 1) // 2

    bufs = ((ii0, jj0, bhi0, bhj0, bxi0, bxj0, gsem0, wsem0),
            (ii1, jj1, bhi1, bhj1, bxi1, bxj1, gsem1, wsem1))

    def valid(t):
        return (t >= 0) & (wid + t * _NW < _NCHS)

    def fire_gather(t, b):
        ii_v, jj_v, bhi, bhj, bxi, bxj, gsem, _ = bufs[b]

        @pl.when(valid(t))
        def _():
            base = (wid + t * _NW) * _CH
            pltpu.sync_copy(ii_hbm.at[pl.ds(base, _CH)], ii_v)
            pltpu.sync_copy(jj_hbm.at[pl.ds(base, _CH)], jj_v)
            pltpu.async_copy(h_hbm.at[ii_v], bhi, gsem)
            pltpu.async_copy(h_hbm.at[jj_v], bhj, gsem)
            pltpu.async_copy(xp_hbm.at[ii_v], bxi, gsem)
            pltpu.async_copy(xp_hbm.at[jj_v], bxj, gsem)

    def drain_gather_fire_write(t, b):
        ii_v, jj_v, bhi, bhj, bxi, bxj, gsem, wsem = bufs[b]

        @pl.when(valid(t))
        def _():
            base = (wid + t * _NW) * _CH
            pltpu.make_async_copy(h_hbm.at[ii_v], bhi, gsem).wait()
            pltpu.make_async_copy(h_hbm.at[jj_v], bhj, gsem).wait()
            pltpu.make_async_copy(xp_hbm.at[ii_v], bxi, gsem).wait()
            pltpu.make_async_copy(xp_hbm.at[jj_v], bxj, gsem).wait()
            pltpu.async_copy(bhi, hi_hbm.at[pl.ds(base, _CH)], wsem)
            pltpu.async_copy(bhj, hj_hbm.at[pl.ds(base, _CH)], wsem)
            pltpu.async_copy(bxi, xi_hbm.at[pl.ds(base, _CH)], wsem)
            pltpu.async_copy(bxj, xj_hbm.at[pl.ds(base, _CH)], wsem)

    def drain_write(t, b):
        _, _, bhi, bhj, bxi, bxj, _, wsem = bufs[b]

        @pl.when(valid(t))
        def _():
            base = (wid + t * _NW) * _CH
            pltpu.make_async_copy(bhi, hi_hbm.at[pl.ds(base, _CH)], wsem).wait()
            pltpu.make_async_copy(bhj, hj_hbm.at[pl.ds(base, _CH)], wsem).wait()
            pltpu.make_async_copy(bxi, xi_hbm.at[pl.ds(base, _CH)], wsem).wait()
            pltpu.make_async_copy(bxj, xj_hbm.at[pl.ds(base, _CH)], wsem).wait()

    def pair(i, carry):
        t0 = 2 * i
        t1 = 2 * i + 1
        drain_write(t0 - 2, 0)
        fire_gather(t0, 0)
        drain_gather_fire_write(t1 - 2, 1)
        drain_write(t1 - 2, 1)
        fire_gather(t1, 1)
        drain_gather_fire_write(t0, 0)
        return carry

    lax.fori_loop(0, pairs, pair, 0)
    last = 2 * pairs - 1
    drain_gather_fire_write(last, 1)
    drain_write(last - 1, 0)
    drain_write(last, 1)


@functools.lru_cache(maxsize=None)
def _gather_kernel():
    return pl.kernel(
        _sc_gather_body,
        out_type=(
            jax.ShapeDtypeStruct((_ES, _ND), jnp.float32),
            jax.ShapeDtypeStruct((_ES, _ND), jnp.float32),
            jax.ShapeDtypeStruct((_ES, _XP), jnp.float32),
            jax.ShapeDtypeStruct((_ES, _XP), jnp.float32),
        ),
        mesh=_sc_mesh(),
        compiler_params=pltpu.CompilerParams(use_tc_tiling_on_sc=False),
        scratch_types=[
            pltpu.VMEM((_CH,), jnp.int32),
            pltpu.VMEM((_CH,), jnp.int32),
            pltpu.VMEM((_CH,), jnp.int32),
            pltpu.VMEM((_CH,), jnp.int32),
            pltpu.VMEM((_CH, _ND), jnp.float32),
            pltpu.VMEM((_CH, _ND), jnp.float32),
            pltpu.VMEM((_CH, _XP), jnp.float32),
            pltpu.VMEM((_CH, _XP), jnp.float32),
            pltpu.VMEM((_CH, _ND), jnp.float32),
            pltpu.VMEM((_CH, _ND), jnp.float32),
            pltpu.VMEM((_CH, _XP), jnp.float32),
            pltpu.VMEM((_CH, _XP), jnp.float32),
            pltpu.SemaphoreType.DMA,
            pltpu.SemaphoreType.DMA,
            pltpu.SemaphoreType.DMA,
            pltpu.SemaphoreType.DMA,
        ],
    )


def _gather_call(h, xp, ii, jj):
    return _gather_kernel()(h, xp, ii, jj)


def _sc_scatter_body(mlo_hbm, mhi_hbm, wd_hbm, ii_hbm, ilo_hbm, ihi_hbm,
                     ix_hbm,
                     silo_hbm, sihi_hbm, xacc_hbm,
                     idx_v, mbuf, wbuf, shm, shx, sem):
    c = lax.axis_index("c")
    s = lax.axis_index("s")
    rows = pl.ds(s * _NPS, _NPS)

    @pl.when(c == 0)
    def _():
        pltpu.sync_copy(ilo_hbm.at[rows], shm.at[rows])
        pltpu.sync_copy(ix_hbm.at[rows], shx.at[rows])

    @pl.when(c == 1)
    def _():
        pltpu.sync_copy(ihi_hbm.at[rows], shm.at[rows])

    plsc.subcore_barrier()

    trips = (_NCHS + _NS - 1) // _NS

    def body(t, carry):
        chunk = s + t * _NS

        @pl.when(chunk < _NCHS)
        def _():
            base = chunk * _CH
            pltpu.sync_copy(ii_hbm.at[pl.ds(base, _CH)], idx_v)

            @pl.when(c == 0)
            def _():
                pltpu.sync_copy(mlo_hbm.at[pl.ds(base, _CH)], mbuf)
                pltpu.sync_copy(wd_hbm.at[pl.ds(base, _CH)], wbuf)
                pltpu.sync_copy(mbuf, shm.at[idx_v], add=True)
                pltpu.sync_copy(wbuf, shx.at[idx_v], add=True)

            @pl.when(c == 1)
            def _():
                pltpu.sync_copy(mhi_hbm.at[pl.ds(base, _CH)], mbuf)
                pltpu.sync_copy(mbuf, shm.at[idx_v], add=True)

        return carry

    lax.fori_loop(0, trips, body, 0)
    plsc.subcore_barrier()

    @pl.when(c == 0)
    def _():
        pltpu.sync_copy(shm.at[rows], silo_hbm.at[rows])
        pltpu.sync_copy(shx.at[rows], xacc_hbm.at[rows])

    @pl.when(c == 1)
    def _():
        pltpu.sync_copy(shm.at[rows], sihi_hbm.at[rows])


@functools.lru_cache(maxsize=None)
def _scatter_kernel():
    return pl.kernel(
        _sc_scatter_body,
        out_type=(
            jax.ShapeDtypeStruct((_N, _ND), jnp.float32),
            jax.ShapeDtypeStruct((_N, _ND), jnp.float32),
            jax.ShapeDtypeStruct((_N, _XP), jnp.float32),
        ),
        mesh=_sc_mesh(),
        compiler_params=pltpu.CompilerParams(use_tc_tiling_on_sc=False),
        scratch_types=[
            pltpu.VMEM((_CH,), jnp.int32),
            pltpu.VMEM((_CH, _ND), jnp.float32),
            pltpu.VMEM((_CH, _XP), jnp.float32),
            pltpu.VMEM_SHARED((_N, _ND), jnp.float32),
            pltpu.VMEM_SHARED((_N, _XP), jnp.float32),
            pltpu.SemaphoreType.DMA,
        ],
    )


def _scatter_call(mlo, mhi, wd, ii, ilo, ihi, ix):
    return _scatter_kernel()(mlo, mhi, wd, ii, ilo, ihi, ix)


def _tc_edge_body(hi, hj, xi, xj, ea,
                  w1, b1, w2, b2, wx1, bx1, wx2,
                  mlo_o, mhi_o, wd_o):
    f32 = jnp.float32
    di = xi[...] - xj[...]
    d2 = jnp.sum(di * di, axis=1, keepdims=True) + 1e-8
    dist = jnp.sqrt(d2)
    centers = lax.broadcasted_iota(jnp.int32, (1, _NRBF), 1).astype(f32) * (
        _CUTOFF / (_NRBF - 1))
    zz = (dist - centers) * (_NRBF / _CUTOFF)
    rbf = jnp.exp(-0.5 * zz * zz)
    msg = jnp.concatenate([hi[...], hj[...], rbf, ea[...]], axis=1)
    pre = jnp.dot(msg, w1[...], preferred_element_type=f32) + b1[...]
    m = _silu(pre)
    m = _silu(jnp.dot(m, w2[...], preferred_element_type=f32) + b2[...])
    t = _silu(jnp.dot(m, wx1[...], preferred_element_type=f32) + bx1[...])
    cw = jnp.dot(t, wx2[...], preferred_element_type=f32)
    mlo_o[...] = m[:, :_ND]
    mhi_o[...] = m[:, _ND:]
    wd_o[...] = di * cw


def _edge_call(hi, hj, xi, xj, ea, w1, b1, w2, b2, wx1, bx1, wx2):
    grid = (_ES // _BE,)
    row = lambda i: (i, 0)
    full = lambda i: (0, 0)
    return pl.pallas_call(
        _tc_edge_body,
        grid=grid,
        in_specs=[
            pl.BlockSpec((_BE, _ND), row),
            pl.BlockSpec((_BE, _ND), row),
            pl.BlockSpec((_BE, _XP), row),
            pl.BlockSpec((_BE, _XP), row),
            pl.BlockSpec((_BE, _ED), row),
            pl.BlockSpec((2 * _ND + _NRBF + _ED, _HD), full),
            pl.BlockSpec((1, _HD), full),
            pl.BlockSpec((_HD, _HD), full),
            pl.BlockSpec((1, _HD), full),
            pl.BlockSpec((_HD, _HD), full),
            pl.BlockSpec((1, _HD), full),
            pl.BlockSpec((_HD, 1), full),
        ],
        out_specs=[
            pl.BlockSpec((_BE, _ND), row),
            pl.BlockSpec((_BE, _ND), row),
            pl.BlockSpec((_BE, _XP), row),
        ],
        out_shape=[
            jax.ShapeDtypeStruct((_ES, _ND), jnp.float32),
            jax.ShapeDtypeStruct((_ES, _ND), jnp.float32),
            jax.ShapeDtypeStruct((_ES, _XP), jnp.float32),
        ],
        compiler_params=pltpu.CompilerParams(
            dimension_semantics=("arbitrary",),
        ),
    )(hi, hj, xi, xj, ea, w1, b1, w2, b2, wx1, bx1, wx2)


def _tc_node_body(h, mlo, mhi, xp, xacc,
                  wh1, bh1, wh2, bh2, g, b,
                  hn_o, xp_o):
    f32 = jnp.float32
    hv = h[...]
    cat = jnp.concatenate([hv, mlo[...], mhi[...]], axis=1)
    pre = jnp.dot(cat, wh1[...], preferred_element_type=f32) + bh1[...]
    u = jnp.dot(_silu(pre), wh2[...], preferred_element_type=f32) + bh2[...]
    hn = hv + u
    mu = jnp.mean(hn, axis=1, keepdims=True)
    var = jnp.mean((hn - mu) * (hn - mu), axis=1, keepdims=True)
    hn_o[...] = (hn - mu) * lax.rsqrt(var + 1e-5) * g[...] + b[...]
    xp_o[...] = xp[...] + xacc[...]


def _node_call(h, mlo, mhi, xp, xacc, wh1, bh1, wh2, bh2, g, b):
    grid = (_N // _BN,)
    row = lambda i: (i, 0)
    full = lambda i: (0, 0)
    return pl.pallas_call(
        _tc_node_body,
        grid=grid,
        in_specs=[
            pl.BlockSpec((_BN, _ND), row),
            pl.BlockSpec((_BN, _ND), row),
            pl.BlockSpec((_BN, _ND), row),
            pl.BlockSpec((_BN, _XP), row),
            pl.BlockSpec((_BN, _XP), row),
            pl.BlockSpec((_ND + _HD, _HD), full),
            pl.BlockSpec((1, _HD), full),
            pl.BlockSpec((_HD, _ND), full),
            pl.BlockSpec((1, _ND), full),
            pl.BlockSpec((1, _ND), full),
            pl.BlockSpec((1, _ND), full),
        ],
        out_specs=[
            pl.BlockSpec((_BN, _ND), row),
            pl.BlockSpec((_BN, _XP), row),
        ],
        out_shape=[
            jax.ShapeDtypeStruct((_N, _ND), jnp.float32),
            jax.ShapeDtypeStruct((_N, _XP), jnp.float32),
        ],
        compiler_params=pltpu.CompilerParams(
            dimension_semantics=("arbitrary",),
        ),
    )(h, mlo, mhi, xp, xacc, wh1, bh1, wh2, bh2, g, b)


def kernel(h, x, edge_index, edge_attr, params):
    ei = edge_index.astype(jnp.int32)
    iis = [lax.slice(ei[1], (k * _ES,), ((k + 1) * _ES,)) for k in range(_S)]
    jjs = [lax.slice(ei[0], (k * _ES,), ((k + 1) * _ES,)) for k in range(_S)]
    eas = [lax.slice(edge_attr, (k * _ES, 0), ((k + 1) * _ES, _ED))
           for k in range(_S)]
    xp = jnp.pad(x.astype(jnp.float32), ((0, 0), (0, _XP - 3)))
    z = jnp.zeros((_N, _ND), jnp.float32)
    zx = jnp.zeros((_N, _XP), jnp.float32)
    for p in params:
        silo, sihi, xacc = z, z, zx
        for k in range(_S):
            hi, hj, xi, xj = _gather_call(h, xp, iis[k], jjs[k])
            mlo, mhi, wd = _edge_call(
                hi, hj, xi, xj, eas[k],
                p["e1"]["w"], p["e1"]["b"][None],
                p["e2"]["w"], p["e2"]["b"][None],
                p["x1"]["w"], p["x1"]["b"][None], p["x2"]["w"],
            )
            silo, sihi, xacc = _scatter_call(
                mlo, mhi, wd, iis[k], silo, sihi, xacc)
        h, xp = _node_call(
            h, silo, sihi, xp, xacc,
            p["h1"]["w"], p["h1"]["b"][None],
            p["h2"]["w"], p["h2"]["b"][None], p["ln_g"][None], p["ln_b"][None],
        )
    return (h, xp[:, :3])


# speedup vs baseline: 2.2191x; 1.0402x over previous
import functools

import jax
import jax.numpy as jnp
from jax import lax
from jax.experimental import pallas as pl
from jax.experimental.pallas import tpu as pltpu
from jax.experimental.pallas import tpu_sc as plsc

_N = 10000
_E = 320000
_ND = 128
_HD = 256
_ED = 16
_NRBF = 16
_CUTOFF = 10.0
_XP = 16

_S = 4
_ES = _E // _S
_CH = 128
_NCHS = _ES // _CH
_NC = 2
_NS = 16
_NW = _NC * _NS
_NPS = _N // _NS

_BE = 1600
_BN = 1000

@functools.lru_cache(maxsize=None)
def _sc_mesh():
    return plsc.VectorSubcoreMesh(
        core_axis_name="c", subcore_axis_name="s", num_cores=_NC, num_subcores=_NS
    )


def _silu(v):
    return v * jax.nn.sigmoid(v)


def _sc_gather_body(h_hbm, xp_hbm, ii_hbm, jj_hbm,
                    hi_hbm, hj_hbm, xi_hbm, xj_hbm,
                    ii0, jj0, ii1, jj1,
                    bhi0, bhj0, bxi0, bxj0,
                    bhi1, bhj1, bxi1, bxj1,
                    gsem0, gsem1, wsem0, wsem1):
    c = lax.axis_index("c")
    s = lax.axis_index("s")
    wid = s * _NC + c
    trips = (_NCHS + _NW - 1) // _NW
    pairs = (trips + 1) // 2

    bufs = ((ii0, jj0, bhi0, bhj0, bxi0, bxj0, gsem0, wsem0),
            (ii1, jj1, bhi1, bhj1, bxi1, bxj1, gsem1, wsem1))

    def valid(t):
        return (t >= 0) & (wid + t * _NW < _NCHS)

    def fire_gather(t, b):
        ii_v, jj_v, bhi, bhj, bxi, bxj, gsem, _ = bufs[b]

        @pl.when(valid(t))
        def _():
            base = (wid + t * _NW) * _CH
            pltpu.sync_copy(ii_hbm.at[pl.ds(base, _CH)], ii_v)
            pltpu.sync_copy(jj_hbm.at[pl.ds(base, _CH)], jj_v)
            pltpu.async_copy(h_hbm.at[ii_v], bhi, gsem)
            pltpu.async_copy(h_hbm.at[jj_v], bhj, gsem)
            pltpu.async_copy(xp_hbm.at[ii_v], bxi, gsem)
            pltpu.async_copy(xp_hbm.at[jj_v], bxj, gsem)

    def drain_gather_fire_write(t, b):
        ii_v, jj_v, bhi, bhj, bxi, bxj, gsem, wsem = bufs[b]

        @pl.when(valid(t))
        def _():
            base = (wid + t * _NW) * _CH
            pltpu.make_async_copy(h_hbm.at[ii_v], bhi, gsem).wait()
            pltpu.make_async_copy(h_hbm.at[jj_v], bhj, gsem).wait()
            pltpu.make_async_copy(xp_hbm.at[ii_v], bxi, gsem).wait()
            pltpu.make_async_copy(xp_hbm.at[jj_v], bxj, gsem).wait()
            pltpu.async_copy(bhi, hi_hbm.at[pl.ds(base, _CH)], wsem)
            pltpu.async_copy(bhj, hj_hbm.at[pl.ds(base, _CH)], wsem)
            pltpu.async_copy(bxi, xi_hbm.at[pl.ds(base, _CH)], wsem)
            pltpu.async_copy(bxj, xj_hbm.at[pl.ds(base, _CH)], wsem)

    def drain_write(t, b):
        _, _, bhi, bhj, bxi, bxj, _, wsem = bufs[b]

        @pl.when(valid(t))
        def _():
            base = (wid + t * _NW) * _CH
            pltpu.make_async_copy(bhi, hi_hbm.at[pl.ds(base, _CH)], wsem).wait()
            pltpu.make_async_copy(bhj, hj_hbm.at[pl.ds(base, _CH)], wsem).wait()
            pltpu.make_async_copy(bxi, xi_hbm.at[pl.ds(base, _CH)], wsem).wait()
            pltpu.make_async_copy(bxj, xj_hbm.at[pl.ds(base, _CH)], wsem).wait()

    def pair(i, carry):
        t0 = 2 * i
        t1 = 2 * i + 1
        drain_write(t0 - 2, 0)
        fire_gather(t0, 0)
        drain_gather_fire_write(t1 - 2, 1)
        drain_write(t1 - 2, 1)
        fire_gather(t1, 1)
        drain_gather_fire_write(t0, 0)
        return carry

    lax.fori_loop(0, pairs, pair, 0)
    last = 2 * pairs - 1
    drain_gather_fire_write(last, 1)
    drain_write(last - 1, 0)
    drain_write(last, 1)


@functools.lru_cache(maxsize=None)
def _gather_kernel():
    return pl.kernel(
        _sc_gather_body,
        out_type=(
            jax.ShapeDtypeStruct((_ES, _ND), jnp.float32),
            jax.ShapeDtypeStruct((_ES, _ND), jnp.float32),
            jax.ShapeDtypeStruct((_ES, _XP), jnp.float32),
            jax.ShapeDtypeStruct((_ES, _XP), jnp.float32),
        ),
        mesh=_sc_mesh(),
        compiler_params=pltpu.CompilerParams(use_tc_tiling_on_sc=False),
        scratch_types=[
            pltpu.VMEM((_CH,), jnp.int32),
            pltpu.VMEM((_CH,), jnp.int32),
            pltpu.VMEM((_CH,), jnp.int32),
            pltpu.VMEM((_CH,), jnp.int32),
            pltpu.VMEM((_CH, _ND), jnp.float32),
            pltpu.VMEM((_CH, _ND), jnp.float32),
            pltpu.VMEM((_CH, _XP), jnp.float32),
            pltpu.VMEM((_CH, _XP), jnp.float32),
            pltpu.VMEM((_CH, _ND), jnp.float32),
            pltpu.VMEM((_CH, _ND), jnp.float32),
            pltpu.VMEM((_CH, _XP), jnp.float32),
            pltpu.VMEM((_CH, _XP), jnp.float32),
            pltpu.SemaphoreType.DMA,
            pltpu.SemaphoreType.DMA,
            pltpu.SemaphoreType.DMA,
            pltpu.SemaphoreType.DMA,
        ],
    )


def _gather_call(h, xp, ii, jj):
    return _gather_kernel()(h, xp, ii, jj)


def _sc_scatter_body(mlo_hbm, mhi_hbm, wd_hbm, ii_hbm, ilo_hbm, ihi_hbm,
                     ix_hbm,
                     silo_hbm, sihi_hbm, xacc_hbm,
                     idx0, idx1, mb0, mb1, wb0, wb1,
                     shm, shx, rsem0, rsem1, ssem0, ssem1):
    c = lax.axis_index("c")
    s = lax.axis_index("s")
    rows = pl.ds(s * _NPS, _NPS)

    @pl.when(c == 0)
    def _():
        pltpu.sync_copy(ilo_hbm.at[rows], shm.at[rows])
        pltpu.sync_copy(ix_hbm.at[rows], shx.at[rows])

    @pl.when(c == 1)
    def _():
        pltpu.sync_copy(ihi_hbm.at[rows], shm.at[rows])

    plsc.subcore_barrier()

    trips = (_NCHS + _NS - 1) // _NS
    pairs = (trips + 1) // 2

    bufs = ((idx0, mb0, wb0, rsem0, ssem0),
            (idx1, mb1, wb1, rsem1, ssem1))

    def valid(t):
        return (t >= 0) & (s + t * _NS < _NCHS)

    def fire_read(t, b):
        idx_v, mb, wb, rsem, _ = bufs[b]
        base = (s + t * _NS) * _CH

        @pl.when(valid(t))
        def _():
            pltpu.async_copy(ii_hbm.at[pl.ds(base, _CH)], idx_v, rsem)

        @pl.when(valid(t) & (c == 0))
        def _():
            pltpu.async_copy(mlo_hbm.at[pl.ds(base, _CH)], mb, rsem)
            pltpu.async_copy(wd_hbm.at[pl.ds(base, _CH)], wb, rsem)

        @pl.when(valid(t) & (c == 1))
        def _():
            pltpu.async_copy(mhi_hbm.at[pl.ds(base, _CH)], mb, rsem)

    def drain_read_fire_add(t, b):
        idx_v, mb, wb, rsem, ssem = bufs[b]
        base = (s + t * _NS) * _CH

        @pl.when(valid(t))
        def _():
            pltpu.make_async_copy(ii_hbm.at[pl.ds(base, _CH)], idx_v,
                                  rsem).wait()

        @pl.when(valid(t) & (c == 0))
        def _():
            pltpu.make_async_copy(mlo_hbm.at[pl.ds(base, _CH)], mb,
                                  rsem).wait()
            pltpu.make_async_copy(wd_hbm.at[pl.ds(base, _CH)], wb,
                                  rsem).wait()
            pltpu.async_copy(mb, shm.at[idx_v], ssem, add=True)
            pltpu.async_copy(wb, shx.at[idx_v], ssem, add=True)

        @pl.when(valid(t) & (c == 1))
        def _():
            pltpu.make_async_copy(mhi_hbm.at[pl.ds(base, _CH)], mb,
                                  rsem).wait()
            pltpu.async_copy(mb, shm.at[idx_v], ssem, add=True)

    def drain_add(t, b):
        idx_v, mb, wb, _, ssem = bufs[b]

        @pl.when(valid(t) & (c == 0))
        def _():
            pltpu.make_async_copy(mb, shm.at[idx_v], ssem).wait()
            pltpu.make_async_copy(wb, shx.at[idx_v], ssem).wait()

        @pl.when(valid(t) & (c == 1))
        def _():
            pltpu.make_async_copy(mb, shm.at[idx_v], ssem).wait()

    def pair(i, carry):
        t0 = 2 * i
        t1 = 2 * i + 1
        drain_add(t0 - 2, 0)
        fire_read(t0, 0)
        drain_read_fire_add(t1 - 2, 1)
        drain_add(t1 - 2, 1)
        fire_read(t1, 1)
        drain_read_fire_add(t0, 0)
        return carry

    lax.fori_loop(0, pairs, pair, 0)
    last = 2 * pairs - 1
    drain_read_fire_add(last, 1)
    drain_add(last - 1, 0)
    drain_add(last, 1)
    plsc.subcore_barrier()

    @pl.when(c == 0)
    def _():
        pltpu.sync_copy(shm.at[rows], silo_hbm.at[rows])
        pltpu.sync_copy(shx.at[rows], xacc_hbm.at[rows])

    @pl.when(c == 1)
    def _():
        pltpu.sync_copy(shm.at[rows], sihi_hbm.at[rows])


@functools.lru_cache(maxsize=None)
def _scatter_kernel():
    return pl.kernel(
        _sc_scatter_body,
        out_type=(
            jax.ShapeDtypeStruct((_N, _ND), jnp.float32),
            jax.ShapeDtypeStruct((_N, _ND), jnp.float32),
            jax.ShapeDtypeStruct((_N, _XP), jnp.float32),
        ),
        mesh=_sc_mesh(),
        compiler_params=pltpu.CompilerParams(use_tc_tiling_on_sc=False),
        scratch_types=[
            pltpu.VMEM((_CH,), jnp.int32),
            pltpu.VMEM((_CH,), jnp.int32),
            pltpu.VMEM((_CH, _ND), jnp.float32),
            pltpu.VMEM((_CH, _ND), jnp.float32),
            pltpu.VMEM((_CH, _XP), jnp.float32),
            pltpu.VMEM((_CH, _XP), jnp.float32),
            pltpu.VMEM_SHARED((_N, _ND), jnp.float32),
            pltpu.VMEM_SHARED((_N, _XP), jnp.float32),
            pltpu.SemaphoreType.DMA,
            pltpu.SemaphoreType.DMA,
            pltpu.SemaphoreType.DMA,
            pltpu.SemaphoreType.DMA,
        ],
    )


def _scatter_call(mlo, mhi, wd, ii, ilo, ihi, ix):
    return _scatter_kernel()(mlo, mhi, wd, ii, ilo, ihi, ix)


def _tc_edge_body(hi, hj, xi, xj, ea,
                  w1, b1, w2, b2, wx1, bx1, wx2,
                  mlo_o, mhi_o, wd_o):
    f32 = jnp.float32
    di = xi[...] - xj[...]
    d2 = jnp.sum(di * di, axis=1, keepdims=True) + 1e-8
    dist = jnp.sqrt(d2)
    centers = lax.broadcasted_iota(jnp.int32, (1, _NRBF), 1).astype(f32) * (
        _CUTOFF / (_NRBF - 1))
    zz = (dist - centers) * (_NRBF / _CUTOFF)
    rbf = jnp.exp(-0.5 * zz * zz)
    msg = jnp.concatenate([hi[...], hj[...], rbf, ea[...]], axis=1)
    pre = jnp.dot(msg, w1[...], preferred_element_type=f32) + b1[...]
    m = _silu(pre)
    m = _silu(jnp.dot(m, w2[...], preferred_element_type=f32) + b2[...])
    t = _silu(jnp.dot(m, wx1[...], preferred_element_type=f32) + bx1[...])
    cw = jnp.dot(t, wx2[...], preferred_element_type=f32)
    mlo_o[...] = m[:, :_ND]
    mhi_o[...] = m[:, _ND:]
    wd_o[...] = di * cw


def _edge_call(hi, hj, xi, xj, ea, w1, b1, w2, b2, wx1, bx1, wx2):
    grid = (_ES // _BE,)
    row = lambda i: (i, 0)
    full = lambda i: (0, 0)
    return pl.pallas_call(
        _tc_edge_body,
        grid=grid,
        in_specs=[
            pl.BlockSpec((_BE, _ND), row),
            pl.BlockSpec((_BE, _ND), row),
            pl.BlockSpec((_BE, _XP), row),
            pl.BlockSpec((_BE, _XP), row),
            pl.BlockSpec((_BE, _ED), row),
            pl.BlockSpec((2 * _ND + _NRBF + _ED, _HD), full),
            pl.BlockSpec((1, _HD), full),
            pl.BlockSpec((_HD, _HD), full),
            pl.BlockSpec((1, _HD), full),
            pl.BlockSpec((_HD, _HD), full),
            pl.BlockSpec((1, _HD), full),
            pl.BlockSpec((_HD, 1), full),
        ],
        out_specs=[
            pl.BlockSpec((_BE, _ND), row),
            pl.BlockSpec((_BE, _ND), row),
            pl.BlockSpec((_BE, _XP), row),
        ],
        out_shape=[
            jax.ShapeDtypeStruct((_ES, _ND), jnp.float32),
            jax.ShapeDtypeStruct((_ES, _ND), jnp.float32),
            jax.ShapeDtypeStruct((_ES, _XP), jnp.float32),
        ],
        compiler_params=pltpu.CompilerParams(
            dimension_semantics=("arbitrary",),
        ),
    )(hi, hj, xi, xj, ea, w1, b1, w2, b2, wx1, bx1, wx2)


def _tc_node_body(h, mlo, mhi, xp, xacc,
                  wh1, bh1, wh2, bh2, g, b,
                  hn_o, xp_o):
    f32 = jnp.float32
    hv = h[...]
    cat = jnp.concatenate([hv, mlo[...], mhi[...]], axis=1)
    pre = jnp.dot(cat, wh1[...], preferred_element_type=f32) + bh1[...]
    u = jnp.dot(_silu(pre), wh2[...], preferred_element_type=f32) + bh2[...]
    hn = hv + u
    mu = jnp.mean(hn, axis=1, keepdims=True)
    var = jnp.mean((hn - mu) * (hn - mu), axis=1, keepdims=True)
    hn_o[...] = (hn - mu) * lax.rsqrt(var + 1e-5) * g[...] + b[...]
    xp_o[...] = xp[...] + xacc[...]


def _node_call(h, mlo, mhi, xp, xacc, wh1, bh1, wh2, bh2, g, b):
    grid = (_N // _BN,)
    row = lambda i: (i, 0)
    full = lambda i: (0, 0)
    return pl.pallas_call(
        _tc_node_body,
        grid=grid,
        in_specs=[
            pl.BlockSpec((_BN, _ND), row),
            pl.BlockSpec((_BN, _ND), row),
            pl.BlockSpec((_BN, _ND), row),
            pl.BlockSpec((_BN, _XP), row),
            pl.BlockSpec((_BN, _XP), row),
            pl.BlockSpec((_ND + _HD, _HD), full),
            pl.BlockSpec((1, _HD), full),
            pl.BlockSpec((_HD, _ND), full),
            pl.BlockSpec((1, _ND), full),
            pl.BlockSpec((1, _ND), full),
            pl.BlockSpec((1, _ND), full),
        ],
        out_specs=[
            pl.BlockSpec((_BN, _ND), row),
            pl.BlockSpec((_BN, _XP), row),
        ],
        out_shape=[
            jax.ShapeDtypeStruct((_N, _ND), jnp.float32),
            jax.ShapeDtypeStruct((_N, _XP), jnp.float32),
        ],
        compiler_params=pltpu.CompilerParams(
            dimension_semantics=("arbitrary",),
        ),
    )(h, mlo, mhi, xp, xacc, wh1, bh1, wh2, bh2, g, b)


def kernel(h, x, edge_index, edge_attr, params):
    ei = edge_index.astype(jnp.int32)
    iis = [lax.slice(ei[1], (k * _ES,), ((k + 1) * _ES,)) for k in range(_S)]
    jjs = [lax.slice(ei[0], (k * _ES,), ((k + 1) * _ES,)) for k in range(_S)]
    eas = [lax.slice(edge_attr, (k * _ES, 0), ((k + 1) * _ES, _ED))
           for k in range(_S)]
    xp = jnp.pad(x.astype(jnp.float32), ((0, 0), (0, _XP - 3)))
    z = jnp.zeros((_N, _ND), jnp.float32)
    zx = jnp.zeros((_N, _XP), jnp.float32)
    for p in params:
        silo, sihi, xacc = z, z, zx
        for k in range(_S):
            hi, hj, xi, xj = _gather_call(h, xp, iis[k], jjs[k])
            mlo, mhi, wd = _edge_call(
                hi, hj, xi, xj, eas[k],
                p["e1"]["w"], p["e1"]["b"][None],
                p["e2"]["w"], p["e2"]["b"][None],
                p["x1"]["w"], p["x1"]["b"][None], p["x2"]["w"],
            )
            silo, sihi, xacc = _scatter_call(
                mlo, mhi, wd, iis[k], silo, sihi, xacc)
        h, xp = _node_call(
            h, silo, sihi, xp, xacc,
            p["h1"]["w"], p["h1"]["b"][None],
            p["h2"]["w"], p["h2"]["b"][None], p["ln_g"][None], p["ln_b"][None],
        )
    return (h, xp[:, :3])
